# baseline (speedup 1.0000x reference)
import functools

import jax
import jax.numpy as jnp
from jax import lax
from jax.experimental import pallas as pl
from jax.experimental.pallas import tpu as pltpu
from jax.experimental.pallas import tpu_sc as plsc

B, L, D = 4096, 200, 128
NC, NS, LANES = 2, 16, 16
NW = NC * NS
RPW = B // NW
NCH = (L + LANES - 1) // LANES


def _sc_body(pos_hbm, ids_hbm, memb_hbm, out_hbm,
             buf0, buf1, ids_v, memb_v,
             insem0, insem1, outsem0, outsem1, small_sem):
    wid = lax.axis_index("s") * NC + lax.axis_index("c")
    base = wid * RPW

    pltpu.make_async_copy(
        ids_hbm.at[pl.ds(base * L, RPW * L)],
        ids_v.at[pl.ds(0, RPW * L)], small_sem).start()
    pltpu.make_async_copy(memb_hbm, memb_v, small_sem).start()
    pltpu.make_async_copy(
        ids_hbm.at[pl.ds(base * L, RPW * L)],
        ids_v.at[pl.ds(0, RPW * L)], small_sem).wait()
    pltpu.make_async_copy(memb_hbm, memb_v, small_sem).wait()

    membc = [memb_v[pl.ds(16 * c, 16)] for c in range(8)]
    lane = lax.iota(jnp.int32, LANES)

    bufs = (buf0, buf1)
    insems = (insem0, insem1)
    outsems = (outsem0, outsem1)

    def start_in(slot, r):
        pltpu.make_async_copy(pos_hbm.at[base + r], bufs[slot],
                              insems[slot]).start()

    def wait_in(slot, r):
        pltpu.make_async_copy(pos_hbm.at[base + r], bufs[slot],
                              insems[slot]).wait()

    def start_out(slot, r):
        pltpu.make_async_copy(bufs[slot], out_hbm.at[base + r],
                              outsems[slot]).start()

    def wait_out(slot, r):
        pltpu.make_async_copy(bufs[slot], out_hbm.at[base + r],
                              outsems[slot]).wait()

    def process(slot, r):
        buf = bufs[slot]
        for c in range(8):
            buf[L - 1, pl.ds(16 * c, 16)] = membc[c]
        zacc = jnp.zeros((LANES,), jnp.int32)
        for k in range(NCH):
            lab = ids_v[pl.ds(r * L + 16 * k + 1, 16)]
            z = lab == 0
            if k == NCH - 1:
                z = z & (lane < (L - 1 - 16 * k))
            zacc = zacc + jnp.where(z, 1, 0)
        for s in (8, 4, 2, 1):
            idx = jnp.bitwise_and(lane + s, LANES - 1)
            zacc = zacc + zacc.at[idx].get(mode="promise_in_bounds")

        @pl.when(zacc[0] > 0)
        def _slow():
            def jbody(j, _):
                idv = ids_v[pl.ds(r * L + j + 1, 16)][0]

                @pl.when(idv == 0)
                def _ow():
                    for c in range(8):
                        buf[j, pl.ds(16 * c, 16)] = membc[c]
                return 0
            lax.fori_loop(0, L - 1, jbody, 0)

    start_in(0, 0)
    start_in(1, 1)

    def loop_body(i, _):
        r0 = 2 * i
        r1 = r0 + 1
        wait_in(0, r0)
        process(0, r0)
        start_out(0, r0)
        wait_in(1, r1)
        process(1, r1)
        start_out(1, r1)

        @pl.when(i < RPW // 2 - 1)
        def _refill():
            wait_out(0, r0)
            start_in(0, r0 + 2)
            wait_out(1, r1)
            start_in(1, r1 + 2)
        return 0

    lax.fori_loop(0, RPW // 2, loop_body, 0)
    wait_out(0, RPW - 2)
    wait_out(1, RPW - 1)


def _sc_out(pos_emb, ids_flat, masked_item_embedding):
    mesh = plsc.VectorSubcoreMesh(core_axis_name="c", subcore_axis_name="s")
    f = pl.kernel(
        _sc_body,
        out_type=jax.ShapeDtypeStruct((B, L, D), jnp.float32),
        mesh=mesh,
        scratch_types=[
            pltpu.VMEM((L, D), jnp.float32),
            pltpu.VMEM((L, D), jnp.float32),
            pltpu.VMEM((RPW * L + 16,), jnp.int32),
            pltpu.VMEM((D,), jnp.float32),
            pltpu.SemaphoreType.DMA,
            pltpu.SemaphoreType.DMA,
            pltpu.SemaphoreType.DMA,
            pltpu.SemaphoreType.DMA,
            pltpu.SemaphoreType.DMA,
        ],
    )
    return f(pos_emb, ids_flat, masked_item_embedding)


def _tc_body(ids_ref, lab_ref, mask_ref):
    ids = ids_ref[...]
    lane = jax.lax.broadcasted_iota(jnp.int32, (B, L), 1)
    labels = jnp.where(lane == (L - 1), 0, jnp.roll(ids, -1, axis=1))
    lab_ref[...] = labels
    mask_ref[...] = labels != 0


def _tc_labels(itemid_seq):
    return pl.pallas_call(
        _tc_body,
        out_shape=[
            jax.ShapeDtypeStruct((B, L), jnp.int32),
            jax.ShapeDtypeStruct((B, L), jnp.bool_),
        ],
    )(itemid_seq)


def kernel(pos_emb, itemid_seq, training, masked_item_embedding):
    del training
    labels, mask = _tc_labels(itemid_seq)
    out = _sc_out(pos_emb, itemid_seq.reshape(-1), masked_item_embedding)
    return (out, labels, mask)


# speedup vs baseline: 1.0174x; 1.0174x over previous
import jax
import jax.numpy as jnp
from jax import lax
from jax.experimental import pallas as pl
from jax.experimental.pallas import tpu as pltpu
from jax.experimental.pallas import tpu_sc as plsc

B, L, D = 4096, 200, 128
NC, NS, LANES = 2, 16, 16
NW = NC * NS
RPW = B // NW
NCH = (L + LANES - 1) // LANES
NSLOT = 4
WROWS = 32
WWORDS = WROWS * L


def _sc_body(pos_hbm, ids_hbm, memb_hbm, out_hbm,
             buf0, buf1, buf2, buf3, ids_w, memb_v,
             insem0, insem1, insem2, insem3,
             outsem0, outsem1, outsem2, outsem3, small_sem):
    wid = lax.axis_index("s") * NC + lax.axis_index("c")
    base = wid * RPW

    pltpu.make_async_copy(memb_hbm, memb_v, small_sem).start()
    pltpu.make_async_copy(memb_hbm, memb_v, small_sem).wait()
    membc = [memb_v[pl.ds(16 * c, 16)] for c in range(8)]
    lane = lax.iota(jnp.int32, LANES)

    bufs = (buf0, buf1, buf2, buf3)
    insems = (insem0, insem1, insem2, insem3)
    outsems = (outsem0, outsem1, outsem2, outsem3)

    def load_window(r):
        pltpu.make_async_copy(
            ids_hbm.at[pl.ds((base + r) * L, WWORDS)],
            ids_w.at[pl.ds(0, WWORDS)], small_sem).start()
        pltpu.make_async_copy(
            ids_hbm.at[pl.ds((base + r) * L, WWORDS)],
            ids_w.at[pl.ds(0, WWORDS)], small_sem).wait()

    def start_in(slot, r):
        pltpu.make_async_copy(pos_hbm.at[base + r], bufs[slot],
                              insems[slot]).start()

    def wait_in(slot, r):
        pltpu.make_async_copy(pos_hbm.at[base + r], bufs[slot],
                              insems[slot]).wait()

    def start_out(slot, r):
        pltpu.make_async_copy(bufs[slot], out_hbm.at[base + r],
                              outsems[slot]).start()

    def wait_out(slot, r):
        pltpu.make_async_copy(bufs[slot], out_hbm.at[base + r],
                              outsems[slot]).wait()

    def process(slot, r):
        buf = bufs[slot]
        lr = lax.bitwise_and(r, WROWS - 1)
        for c in range(8):
            buf[L - 1, pl.ds(16 * c, 16)] = membc[c]
        zacc = jnp.zeros((LANES,), jnp.int32)
        for k in range(NCH):
            lab = ids_w[pl.ds(lr * L + 16 * k + 1, 16)]
            z = lab == 0
            if k == NCH - 1:
                z = z & (lane < (L - 1 - 16 * k))
            zacc = zacc + jnp.where(z, 1, 0)
        for s in (8, 4, 2, 1):
            idx = jnp.bitwise_and(lane + s, LANES - 1)
            zacc = zacc + zacc.at[idx].get(mode="promise_in_bounds")

        @pl.when(zacc[0] > 0)
        def _slow():
            def jbody(j, _):
                idv = ids_w[pl.ds(lr * L + j + 1, 16)][0]

                @pl.when(idv == 0)
                def _ow():
                    for c in range(8):
                        buf[j, pl.ds(16 * c, 16)] = membc[c]
                return 0
            lax.fori_loop(0, L - 1, jbody, 0)

    for s in range(NSLOT):
        start_in(s, s)
    load_window(0)

    def loop_body(i, _):
        r0 = NSLOT * i
        for s in range(NSLOT):
            r = r0 + s
            if s == 0:
                @pl.when(lax.bitwise_and(r0, WROWS - 1) == 0)
                def _refresh():
                    load_window(r0)
            wait_in(s, r)
            process(s, r)
            start_out(s, r)

            @pl.when(r + NSLOT < RPW)
            def _refill():
                wait_out(s, r)
                start_in(s, r + NSLOT)
        return 0

    lax.fori_loop(0, RPW // NSLOT, loop_body, 0)
    for s in range(NSLOT):
        wait_out(s, RPW - NSLOT + s)


def _sc_out(pos_emb, ids_flat, masked_item_embedding):
    mesh = plsc.VectorSubcoreMesh(core_axis_name="c", subcore_axis_name="s")
    f = pl.kernel(
        _sc_body,
        out_type=jax.ShapeDtypeStruct((B, L, D), jnp.float32),
        mesh=mesh,
        scratch_types=[
            pltpu.VMEM((L, D), jnp.float32),
            pltpu.VMEM((L, D), jnp.float32),
            pltpu.VMEM((L, D), jnp.float32),
            pltpu.VMEM((L, D), jnp.float32),
            pltpu.VMEM((WWORDS + 16,), jnp.int32),
            pltpu.VMEM((D,), jnp.float32),
            pltpu.SemaphoreType.DMA,
            pltpu.SemaphoreType.DMA,
            pltpu.SemaphoreType.DMA,
            pltpu.SemaphoreType.DMA,
            pltpu.SemaphoreType.DMA,
            pltpu.SemaphoreType.DMA,
            pltpu.SemaphoreType.DMA,
            pltpu.SemaphoreType.DMA,
            pltpu.SemaphoreType.DMA,
        ],
    )
    return f(pos_emb, ids_flat, masked_item_embedding)


def _tc_body(ids_ref, lab_ref, mask_ref):
    ids = ids_ref[...]
    lane = jax.lax.broadcasted_iota(jnp.int32, (B, L), 1)
    labels = jnp.where(lane == (L - 1), 0, jnp.roll(ids, -1, axis=1))
    lab_ref[...] = labels
    mask_ref[...] = labels != 0


def _tc_labels(itemid_seq):
    return pl.pallas_call(
        _tc_body,
        out_shape=[
            jax.ShapeDtypeStruct((B, L), jnp.int32),
            jax.ShapeDtypeStruct((B, L), jnp.bool_),
        ],
    )(itemid_seq)


def kernel(pos_emb, itemid_seq, training, masked_item_embedding):
    del training
    labels, mask = _tc_labels(itemid_seq)
    out = _sc_out(pos_emb, itemid_seq.reshape(-1), masked_item_embedding)
    return (out, labels, mask)


# speedup vs baseline: 1.1523x; 1.1326x over previous
import jax
import jax.numpy as jnp
from jax.experimental import pallas as pl
from jax.experimental.pallas import tpu as pltpu

B, L, D = 4096, 200, 128
BB = 64


def _body(ids_ref, pos_ref, memb_ref, out_ref, lab_ref, mask_ref):
    ids = ids_ref[...]
    lane = jax.lax.broadcasted_iota(jnp.int32, (BB, L), 1)
    labels = jnp.where(lane == (L - 1), 0, jnp.roll(ids, -1, axis=1))
    lab_ref[...] = labels
    mask_ref[...] = labels != 0

    memb = memb_ref[...]

    out_ref[...] = pos_ref[...]
    out_ref[:, L - 1, :] = jnp.broadcast_to(memb[0], (BB, D))

    anyz = jnp.any(jnp.logical_and(labels == 0, lane < (L - 1)))

    @pl.when(anyz)
    def _slow():
        labels3 = jnp.transpose(labels.reshape(BB, 1, L), (0, 2, 1))
        out_ref[...] = jnp.where(labels3 != 0, pos_ref[...], memb)


def kernel(pos_emb, itemid_seq, training, masked_item_embedding):
    del training
    memb3 = masked_item_embedding.reshape(1, 1, D)
    grid = (B // BB,)
    out, labels, mask = pl.pallas_call(
        _body,
        grid=grid,
        in_specs=[
            pl.BlockSpec((BB, L), lambda i: (i, 0)),
            pl.BlockSpec((BB, L, D), lambda i: (i, 0, 0)),
            pl.BlockSpec((1, 1, D), lambda i: (0, 0, 0)),
        ],
        out_specs=[
            pl.BlockSpec((BB, L, D), lambda i: (i, 0, 0)),
            pl.BlockSpec((BB, L), lambda i: (i, 0)),
            pl.BlockSpec((BB, L), lambda i: (i, 0)),
        ],
        out_shape=[
            jax.ShapeDtypeStruct((B, L, D), jnp.float32),
            jax.ShapeDtypeStruct((B, L), jnp.int32),
            jax.ShapeDtypeStruct((B, L), jnp.bool_),
        ],
    )(itemid_seq, pos_emb, memb3)
    return (out, labels, mask)


# speedup vs baseline: 1.2102x; 1.0502x over previous
import jax
import jax.numpy as jnp
from jax.experimental import pallas as pl
from jax.experimental.pallas import tpu as pltpu

B, L, D = 4096, 200, 128
BB = 128


def _body(ids_ref, pos_ref, memb_ref, out_ref, lab_ref, mask_ref):
    ids = ids_ref[...]
    lane = jax.lax.broadcasted_iota(jnp.int32, (BB, L), 1)
    labels = jnp.where(lane == (L - 1), 0, jnp.roll(ids, -1, axis=1))
    lab_ref[...] = labels
    mask_ref[...] = labels != 0

    memb = memb_ref[...]

    out_ref[...] = pos_ref[...]
    out_ref[:, L - 1, :] = jnp.broadcast_to(memb[0], (BB, D))

    HB = BB // 2
    for h in range(2):
        labh = labels[h * HB:(h + 1) * HB]
        anyz = jnp.any(jnp.logical_and(labh == 0, lane[:HB] < (L - 1)))

        @pl.when(anyz)
        def _slow(labh=labh, h=h):
            labels3 = jnp.transpose(labh.reshape(HB, 1, L), (0, 2, 1))
            sl = pl.ds(h * HB, HB)
            out_ref[sl] = jnp.where(labels3 != 0, pos_ref[sl], memb)


def kernel(pos_emb, itemid_seq, training, masked_item_embedding):
    del training
    memb3 = masked_item_embedding.reshape(1, 1, D)
    grid = (B // BB,)
    out, labels, mask = pl.pallas_call(
        _body,
        grid=grid,
        in_specs=[
            pl.BlockSpec((BB, L), lambda i: (i, 0)),
            pl.BlockSpec((BB, L, D), lambda i: (i, 0, 0)),
            pl.BlockSpec((1, 1, D), lambda i: (0, 0, 0)),
        ],
        out_specs=[
            pl.BlockSpec((BB, L, D), lambda i: (i, 0, 0)),
            pl.BlockSpec((BB, L), lambda i: (i, 0)),
            pl.BlockSpec((BB, L), lambda i: (i, 0)),
        ],
        out_shape=[
            jax.ShapeDtypeStruct((B, L, D), jnp.float32),
            jax.ShapeDtypeStruct((B, L), jnp.int32),
            jax.ShapeDtypeStruct((B, L), jnp.bool_),
        ],
    )(itemid_seq, pos_emb, memb3)
    return (out, labels, mask)
